# baseline (speedup 1.0000x reference)
import functools

import jax
import jax.numpy as jnp
from jax import lax
from jax.experimental import pallas as pl
from jax.experimental.pallas import tpu as pltpu
from jax.experimental.pallas import tpu_sc as plsc

B = 1024
W = 20
E = 64
V = 100000

_NC = 2
_NS = 16
_NW = _NC * _NS
_BPW = B // _NW
_LANES = E // 16

BLOCK_V = 4096
_NBLK = (V + BLOCK_V - 1) // BLOCK_V


BLOCK_T = 2048
_NBLK_T = (V + BLOCK_T - 1) // BLOCK_T


def _tpad_body(e_ref, o_ref):
    eye = jnp.eye(E, dtype=jnp.float32)
    et = lax.dot_general(
        e_ref[...], eye, (((0,), (0,)), ((), ())),
        preferred_element_type=jnp.float32,
    )
    o_ref[:, 0:E] = et


def _gather_mean_body(idx_hbm, table_hbm, h_hbm, idx_v, rows_v, hsum_v, sem):
    wid = lax.axis_index("s") * _NC + lax.axis_index("c")
    base_b = wid * _BPW
    pltpu.sync_copy(idx_hbm.at[:, pl.ds(base_b, _BPW)], idx_v)
    copies = [
        pltpu.async_copy(table_hbm.at[idx_v.at[w]], rows_v.at[w], sem)
        for w in range(W)
    ]
    for c in copies:
        c.wait()

    inv_w = jnp.float32(1.0 / W)

    def row_body(b, carry):
        def w_body(w, accs):
            return tuple(
                accs[c] + rows_v[w, b, pl.ds(c * 16, 16)] for c in range(_LANES)
            )

        accs = lax.fori_loop(
            0, W, w_body, tuple(jnp.zeros((16,), jnp.float32) for _ in range(_LANES))
        )
        for c in range(_LANES):
            hsum_v[b, pl.ds(c * 16, 16)] = accs[c] * inv_w
        return carry

    lax.fori_loop(0, _BPW, row_body, 0)
    pltpu.sync_copy(hsum_v, h_hbm.at[pl.ds(base_b, _BPW)])


@functools.lru_cache(maxsize=1)
def _gather_mean():
    return pl.kernel(
        _gather_mean_body,
        out_type=jax.ShapeDtypeStruct((B, E), jnp.float32),
        mesh=plsc.VectorSubcoreMesh(core_axis_name="c", subcore_axis_name="s"),
        scratch_types=[
            pltpu.VMEM((W, _BPW), jnp.int32),
            pltpu.VMEM((W, _BPW, 128), jnp.float32),
            pltpu.VMEM((_BPW, E), jnp.float32),
            pltpu.SemaphoreType.DMA,
        ],
        compiler_params=pltpu.CompilerParams(use_tc_tiling_on_sc=False),
    )


def _mm_body(h_ref, w_ref, b_ref, o_ref):
    acc = lax.dot_general(
        w_ref[...],
        h_ref[...],
        (((0,), (1,)), ((), ())),
        preferred_element_type=jnp.float32,
    )
    ones = jnp.ones((B, 1), jnp.float32)
    o_ref[...] = acc + lax.dot_general(
        b_ref[...],
        ones,
        (((0,), (1,)), ((), ())),
        preferred_element_type=jnp.float32,
    )


def kernel(inputs, embedding, linear_w, linear_b):
    idx_t = inputs.astype(jnp.int32).T
    emb_pad = pl.pallas_call(
        _tpad_body,
        grid=(_NBLK_T,),
        in_specs=[pl.BlockSpec((E, BLOCK_T), lambda i: (0, i))],
        out_specs=pl.BlockSpec((BLOCK_T, 128), lambda i: (i, 0)),
        out_shape=jax.ShapeDtypeStruct((V, 128), jnp.float32),
    )(embedding.T)
    h = _gather_mean()(idx_t, emb_pad)
    w_t = linear_w.T
    b2 = linear_b.reshape(1, V)
    out_t = pl.pallas_call(
        _mm_body,
        grid=(_NBLK,),
        in_specs=[
            pl.BlockSpec((B, E), lambda i: (0, 0)),
            pl.BlockSpec((E, BLOCK_V), lambda i: (0, i)),
            pl.BlockSpec((1, BLOCK_V), lambda i: (0, i)),
        ],
        out_specs=pl.BlockSpec((BLOCK_V, B), lambda i: (i, 0)),
        out_shape=jax.ShapeDtypeStruct((V, B), jnp.float32),
    )(h, w_t, b2)
    return out_t.T


# speedup vs baseline: 1.1056x; 1.1056x over previous
import functools

import jax
import jax.numpy as jnp
from jax import lax
from jax.experimental import pallas as pl
from jax.experimental.pallas import tpu as pltpu
from jax.experimental.pallas import tpu_sc as plsc

B = 1024
W = 20
E = 64
V = 100000

_NC = 2
_NS = 16
_NW = _NC * _NS
_BPW = B // _NW
_LANES = E // 16

BLOCK_V = 4096
_NBLK = (V + BLOCK_V - 1) // BLOCK_V


BLOCK_T = 8192
_NBLK_T = (V + BLOCK_T - 1) // BLOCK_T


def _tpad_body(e_ref, o_ref):
    eye = jnp.eye(E, dtype=jnp.float32)
    et = lax.dot_general(
        e_ref[...], eye, (((0,), (0,)), ((), ())),
        preferred_element_type=jnp.float32,
    )
    o_ref[:, 0:E] = et


def _gather_mean_body(idx_hbm, table_hbm, h_hbm, idx_v, rows_v, hsum_v, sem):
    wid = lax.axis_index("s") * _NC + lax.axis_index("c")
    base_b = wid * _BPW
    pltpu.sync_copy(idx_hbm.at[:, pl.ds(base_b, _BPW)], idx_v)
    copies = [
        pltpu.async_copy(table_hbm.at[idx_v.at[w]], rows_v.at[w], sem)
        for w in range(W)
    ]
    for c in copies:
        c.wait()

    inv_w = jnp.float32(1.0 / W)

    def row_body(b, carry):
        def w_body(w, accs):
            return tuple(
                accs[c] + rows_v[w, b, pl.ds(c * 16, 16)] for c in range(_LANES)
            )

        accs = lax.fori_loop(
            0, W, w_body, tuple(jnp.zeros((16,), jnp.float32) for _ in range(_LANES))
        )
        for c in range(_LANES):
            hsum_v[b, pl.ds(c * 16, 16)] = accs[c] * inv_w
        return carry

    lax.fori_loop(0, _BPW, row_body, 0)
    pltpu.sync_copy(hsum_v, h_hbm.at[pl.ds(base_b, _BPW)])


@functools.lru_cache(maxsize=1)
def _gather_mean():
    return pl.kernel(
        _gather_mean_body,
        out_type=jax.ShapeDtypeStruct((B, E), jnp.float32),
        mesh=plsc.VectorSubcoreMesh(core_axis_name="c", subcore_axis_name="s"),
        scratch_types=[
            pltpu.VMEM((W, _BPW), jnp.int32),
            pltpu.VMEM((W, _BPW, 128), jnp.float32),
            pltpu.VMEM((_BPW, E), jnp.float32),
            pltpu.SemaphoreType.DMA,
        ],
        compiler_params=pltpu.CompilerParams(use_tc_tiling_on_sc=False),
    )


def _mm_body(h_ref, w_ref, b_ref, o_ref):
    acc = lax.dot_general(
        w_ref[...],
        h_ref[...],
        (((0,), (1,)), ((), ())),
        preferred_element_type=jnp.float32,
    )
    ones = jnp.ones((B, 1), jnp.float32)
    o_ref[...] = acc + lax.dot_general(
        b_ref[...],
        ones,
        (((0,), (1,)), ((), ())),
        preferred_element_type=jnp.float32,
    )


def kernel(inputs, embedding, linear_w, linear_b):
    idx_t = inputs.astype(jnp.int32).T
    emb_pad = pl.pallas_call(
        _tpad_body,
        grid=(_NBLK_T,),
        in_specs=[pl.BlockSpec((E, BLOCK_T), lambda i: (0, i))],
        out_specs=pl.BlockSpec((BLOCK_T, 128), lambda i: (i, 0)),
        out_shape=jax.ShapeDtypeStruct((V, 128), jnp.float32),
    )(embedding.T)
    h = _gather_mean()(idx_t, emb_pad)
    w_t = linear_w.T
    b2 = linear_b.reshape(1, V)
    out_t = pl.pallas_call(
        _mm_body,
        grid=(_NBLK,),
        in_specs=[
            pl.BlockSpec((B, E), lambda i: (0, 0)),
            pl.BlockSpec((E, BLOCK_V), lambda i: (0, i)),
            pl.BlockSpec((1, BLOCK_V), lambda i: (0, i)),
        ],
        out_specs=pl.BlockSpec((BLOCK_V, B), lambda i: (i, 0)),
        out_shape=jax.ShapeDtypeStruct((V, B), jnp.float32),
    )(h, w_t, b2)
    return out_t.T


# speedup vs baseline: 1.1219x; 1.0147x over previous
import functools

import jax
import jax.numpy as jnp
from jax import lax
from jax.experimental import pallas as pl
from jax.experimental.pallas import tpu as pltpu
from jax.experimental.pallas import tpu_sc as plsc

B = 1024
W = 20
E = 64
V = 100000

_NC = 2
_NS = 16
_NW = _NC * _NS
_BPW = B // _NW
_LANES = E // 16

BLOCK_V = 4096
_NBLK = (V + BLOCK_V - 1) // BLOCK_V


BLOCK_T = 16384
_NBLK_T = (V + BLOCK_T - 1) // BLOCK_T


def _tpad_body(e_ref, o_ref):
    eye = jnp.eye(E, dtype=jnp.float32)
    et = lax.dot_general(
        e_ref[...], eye, (((0,), (0,)), ((), ())),
        preferred_element_type=jnp.float32,
    )
    o_ref[:, 0:E] = et


def _gather_mean_body(idx_hbm, table_hbm, h_hbm, idx_v, rows_v, hsum_v, sem):
    wid = lax.axis_index("s") * _NC + lax.axis_index("c")
    base_b = wid * _BPW
    pltpu.sync_copy(idx_hbm.at[:, pl.ds(base_b, _BPW)], idx_v)
    copies = [
        pltpu.async_copy(table_hbm.at[idx_v.at[w]], rows_v.at[w], sem)
        for w in range(W)
    ]
    for c in copies:
        c.wait()

    inv_w = jnp.float32(1.0 / W)

    def row_body(b, carry):
        def w_body(w, accs):
            return tuple(
                accs[c] + rows_v[w, b, pl.ds(c * 16, 16)] for c in range(_LANES)
            )

        accs = lax.fori_loop(
            0, W, w_body, tuple(jnp.zeros((16,), jnp.float32) for _ in range(_LANES))
        )
        for c in range(_LANES):
            hsum_v[b, pl.ds(c * 16, 16)] = accs[c] * inv_w
        return carry

    lax.fori_loop(0, _BPW, row_body, 0)
    pltpu.sync_copy(hsum_v, h_hbm.at[pl.ds(base_b, _BPW)])


@functools.lru_cache(maxsize=1)
def _gather_mean():
    return pl.kernel(
        _gather_mean_body,
        out_type=jax.ShapeDtypeStruct((B, E), jnp.float32),
        mesh=plsc.VectorSubcoreMesh(core_axis_name="c", subcore_axis_name="s"),
        scratch_types=[
            pltpu.VMEM((W, _BPW), jnp.int32),
            pltpu.VMEM((W, _BPW, 128), jnp.float32),
            pltpu.VMEM((_BPW, E), jnp.float32),
            pltpu.SemaphoreType.DMA,
        ],
        compiler_params=pltpu.CompilerParams(use_tc_tiling_on_sc=False),
    )


def _mm_body(h_ref, w_ref, b_ref, o_ref):
    acc = lax.dot_general(
        w_ref[...],
        h_ref[...],
        (((0,), (1,)), ((), ())),
        preferred_element_type=jnp.float32,
    )
    ones = jnp.ones((B, 1), jnp.float32)
    o_ref[...] = acc + lax.dot_general(
        b_ref[...],
        ones,
        (((0,), (1,)), ((), ())),
        preferred_element_type=jnp.float32,
    )


def kernel(inputs, embedding, linear_w, linear_b):
    idx_t = inputs.astype(jnp.int32).T
    emb_pad = pl.pallas_call(
        _tpad_body,
        grid=(_NBLK_T,),
        in_specs=[pl.BlockSpec((E, BLOCK_T), lambda i: (0, i))],
        out_specs=pl.BlockSpec((BLOCK_T, 128), lambda i: (i, 0)),
        out_shape=jax.ShapeDtypeStruct((V, 128), jnp.float32),
    )(embedding.T)
    h = _gather_mean()(idx_t, emb_pad)
    w_t = linear_w.T
    b2 = linear_b.reshape(1, V)
    out_t = pl.pallas_call(
        _mm_body,
        grid=(_NBLK,),
        in_specs=[
            pl.BlockSpec((B, E), lambda i: (0, 0)),
            pl.BlockSpec((E, BLOCK_V), lambda i: (0, i)),
            pl.BlockSpec((1, BLOCK_V), lambda i: (0, i)),
        ],
        out_specs=pl.BlockSpec((BLOCK_V, B), lambda i: (i, 0)),
        out_shape=jax.ShapeDtypeStruct((V, B), jnp.float32),
    )(h, w_t, b2)
    return out_t.T
